# Initial kernel scaffold; baseline (speedup 1.0000x reference)
import jax
import jax.numpy as jnp
from jax.experimental import pallas as pl


def kernel(inputs, embedding):
    raise NotImplementedError("write your pallas kernel here")


# speedup vs baseline: 1.1704x; 1.1704x over previous
import functools

import jax
import jax.numpy as jnp
from jax import lax
from jax.experimental import pallas as pl
from jax.experimental.pallas import tpu as pltpu
from jax.experimental.pallas import tpu_sc as plsc

N = 100000
D = 128
CHUNK = 400
G = CHUNK // 16
NCHUNK = N // CHUNK
NC, NS = 2, 16
NW = NC * NS


def _body(idx_hbm, emb_hbm, out_hbm, idx_v, rows_v, sem):
    c = lax.axis_index("c")
    s = lax.axis_index("s")
    wid = s * NC + c
    nloop = (NCHUNK - wid + NW - 1) // NW

    def step(i, carry):
        k = wid + i * NW
        pltpu.sync_copy(idx_hbm.at[k], idx_v)
        for g in range(G):
            sl = pl.ds(g * 16, 16)
            idx_v[sl] = idx_v[sl] - 1
        pltpu.async_copy(emb_hbm.at[idx_v], rows_v, sem).wait()
        pltpu.sync_copy(rows_v, out_hbm.at[k])
        return carry

    lax.fori_loop(0, nloop, step, 0)


@jax.jit
def _embed(idx3, embedding):
    mesh = plsc.VectorSubcoreMesh(core_axis_name="c", subcore_axis_name="s")
    f = functools.partial(
        pl.kernel,
        out_type=jax.ShapeDtypeStruct((NCHUNK, CHUNK, D), jnp.float32),
        mesh=mesh,
        scratch_types=[
            pltpu.VMEM((CHUNK,), jnp.int32),
            pltpu.VMEM((CHUNK, D), jnp.float32),
            pltpu.SemaphoreType.DMA,
        ],
    )(_body)
    return f(idx3, embedding)


def kernel(inputs, embedding):
    idx3 = inputs.reshape(NCHUNK, CHUNK)
    out = _embed(idx3, embedding)
    return out.reshape(N, D)
